# Initial kernel scaffold; baseline (speedup 1.0000x reference)
import jax
import jax.numpy as jnp
from jax.experimental import pallas as pl


def kernel(h, e, edge_index, snorm_n, snorm_e, emb_h, emb_e, We, be, Wp, bp, gamma, beta, Wr0, br0, Wr1, br1, Wr2, br2):
    raise NotImplementedError("write your pallas kernel here")


# speedup vs baseline: 1.8180x; 1.8180x over previous
import functools

import jax
import jax.numpy as jnp
from jax import lax
from jax.experimental import pallas as pl
from jax.experimental.pallas import tpu as pltpu
from jax.experimental.pallas import tpu_sc as plsc

N = 10000
E = 320000
HID = 128
EDIM = 16
L = 4
NUM_ATOM = 100
NUM_BOND = 4
AVG_D_LOG = 3.4965

NW = 32
NPW = 158
NR = 64
NPAD = NR * NPW
CH = 32
BN = 128
NB = NPAD // BN
DUMMY = NPW * HID


def _qdot(x, w):
    xq = x.astype(jnp.bfloat16).astype(jnp.float32)
    wq = w.astype(jnp.bfloat16).astype(jnp.float32)
    return jnp.dot(xq, wq, preferred_element_type=jnp.float32,
                   precision=lax.Precision.HIGHEST)


def _sc_body(h_hbm, codes_hbm, srcs_hbm, est_hbm, tab_hbm,
             out_sum, out_sq, out_mx, out_mn, out_cnt,
             acc_sum, acc_sq, acc_mx, acc_mn, acc_cnt,
             idx_v, code_v, rows_v, est_v, tab_v, sem):
    cid = lax.axis_index("c")
    sid = lax.axis_index("s")
    wid = sid * 2 + cid

    pltpu.sync_copy(tab_hbm, tab_v)

    zero16 = jnp.zeros((16,), jnp.float32)
    ones16 = jnp.ones((16,), jnp.float32)
    neg16 = jnp.full((16,), -1e30, jnp.float32)
    pos16 = jnp.full((16,), 1e30, jnp.float32)
    iota16 = jnp.arange(16, dtype=jnp.int32)

    for p in range(2):
        r = p * NW + wid
        pltpu.sync_copy(est_hbm.at[pl.ds(r * 16, 16)], est_v)
        ev = est_v[pl.ds(0, 16)]
        e_lo = ev[0]
        e_hi = ev[1]
        a_lo = (e_lo // 8) * 8
        nch = (e_hi - a_lo + CH - 1) // CH

        def init_body(i, _):
            off = i * 16
            acc_sum[pl.ds(off, 16)] = zero16
            acc_sq[pl.ds(off, 16)] = zero16
            acc_mx[pl.ds(off, 16)] = neg16
            acc_mn[pl.ds(off, 16)] = pos16
            return 0
        lax.fori_loop(0, (NPW + 1) * 8, init_body, 0)

        def initc_body(i, _):
            acc_cnt[pl.ds(i * 16, 16)] = zero16
            return 0
        lax.fori_loop(0, NPW + 1, initc_body, 0)

        def chunk_body(k, _):
            cstart = a_lo + k * CH
            pltpu.sync_copy(codes_hbm.at[pl.ds(cstart, CH)], code_v)
            pltpu.sync_copy(srcs_hbm.at[pl.ds(cstart, CH)], idx_v)
            pltpu.async_copy(h_hbm.at[idx_v], rows_v, sem).wait()
            for g in range(CH // 16):
                code16 = code_v[pl.ds(g * 16, 16)]
                jg = cstart + g * 16 + iota16
                vmask = (jg >= e_lo) & (jg < e_hi)
                aoff16 = jnp.where(vmask, (code16 >> 2) << 7, DUMMY)
                eoff16 = (code16 & 3) << 7
                for lane in range(16):
                    ao = aoff16[lane]
                    eo = eoff16[lane]
                    j = g * 16 + lane
                    plsc.addupdate(acc_cnt.at[pl.ds((ao >> 3), 16)], ones16)
                    for v in range(8):
                        row = rows_v[j, pl.ds(v * 16, 16)]
                        m = row + tab_v[pl.ds(eo + v * 16, 16)]
                        av = ao + v * 16
                        plsc.addupdate(acc_sum.at[pl.ds(av, 16)], m)
                        plsc.addupdate(acc_sq.at[pl.ds(av, 16)], m * m)
                        acc_mx[pl.ds(av, 16)] = jnp.maximum(
                            acc_mx[pl.ds(av, 16)], m)
                        acc_mn[pl.ds(av, 16)] = jnp.minimum(
                            acc_mn[pl.ds(av, 16)], m)
            return 0
        lax.fori_loop(0, nch, chunk_body, 0)

        npwh = NPW * HID
        base = r * npwh
        pltpu.sync_copy(acc_sum.at[pl.ds(0, npwh)],
                        out_sum.at[pl.ds(base, npwh)])
        pltpu.sync_copy(acc_sq.at[pl.ds(0, npwh)],
                        out_sq.at[pl.ds(base, npwh)])
        pltpu.sync_copy(acc_mx.at[pl.ds(0, npwh)],
                        out_mx.at[pl.ds(base, npwh)])
        pltpu.sync_copy(acc_mn.at[pl.ds(0, npwh)],
                        out_mn.at[pl.ds(base, npwh)])
        pltpu.sync_copy(acc_cnt.at[pl.ds(0, NPW * 16)],
                        out_cnt.at[pl.ds(r * NPW * 16, NPW * 16)])


def _sc_aggregate(h_n, codes_p, srcs_p, est, tab_i):
    mesh = plsc.VectorSubcoreMesh(core_axis_name="c", subcore_axis_name="s")
    f32 = jnp.float32
    agg_t = jax.ShapeDtypeStruct((NPAD * HID,), f32)
    kern = pl.kernel(
        _sc_body,
        mesh=mesh,
        out_type=[agg_t, agg_t, agg_t, agg_t,
                  jax.ShapeDtypeStruct((NPAD * 16,), f32)],
        scratch_types=[
            pltpu.VMEM(((NPW + 1) * HID,), f32),
            pltpu.VMEM(((NPW + 1) * HID,), f32),
            pltpu.VMEM(((NPW + 1) * HID,), f32),
            pltpu.VMEM(((NPW + 1) * HID,), f32),
            pltpu.VMEM(((NPW + 1) * 16,), f32),
            pltpu.VMEM((CH,), jnp.int32),
            pltpu.VMEM((CH,), jnp.int32),
            pltpu.VMEM((CH, HID), f32),
            pltpu.VMEM((16,), jnp.int32),
            pltpu.VMEM((NUM_BOND * HID,), f32),
            pltpu.SemaphoreType.DMA,
        ],
    )
    return kern(h_n, codes_p, srcs_p, est, tab_i)


def _embed_body(hid_ref, emb_ref, h_ref):
    ids = hid_ref[0, 0, :]
    col = lax.broadcasted_iota(jnp.int32, (BN, 128), 1)
    oh = jnp.where(ids[:, None] == col, 1.0, 0.0).astype(jnp.float32)
    h_ref[...] = jnp.dot(oh, emb_ref[...], preferred_element_type=jnp.float32,
                     precision=lax.Precision.HIGHEST)


def _embed_nodes(h_idx3, emb_h_p):
    return pl.pallas_call(
        _embed_body,
        grid=(NB,),
        in_specs=[
            pl.BlockSpec((1, 1, BN), lambda b: (b, 0, 0)),
            pl.BlockSpec((128, 128), lambda b: (0, 0)),
        ],
        out_specs=pl.BlockSpec((BN, HID), lambda b: (b, 0)),
        out_shape=jax.ShapeDtypeStruct((NPAD, HID), jnp.float32),
    )(h_idx3, emb_h_p)


def _tab_body(embe_ref, We_ref, be_ref, tab_ref):
    embe = embe_ref[...]
    for i in range(L):
        w = We_ref[pl.ds(i * EDIM, EDIM), :]
        t = _qdot(embe, w)
        tab_ref[i, :, :] = t + be_ref[i, :][None, :]


def _edge_tables(emb_e_p, We_r, be):
    return pl.pallas_call(
        _tab_body,
        grid=(1,),
        in_specs=[
            pl.BlockSpec((8, EDIM), lambda b: (0, 0)),
            pl.BlockSpec((L * EDIM, HID), lambda b: (0, 0)),
            pl.BlockSpec((L, HID), lambda b: (0, 0)),
        ],
        out_specs=pl.BlockSpec((L, 8, HID), lambda b: (0, 0, 0)),
        out_shape=jax.ShapeDtypeStruct((L, 8, HID), jnp.float32),
    )(emb_e_p, We_r, be)


def _layer1_body(h_ref, sm_ref, sq_ref, mx_ref, mn_ref,
                 cnt_ref, wp_ref, bp_ref, sn_ref, ht_ref, st_ref):
    b = pl.program_id(0)
    h = h_ref[...]
    sm = sm_ref[...]
    sq = sq_ref[...]
    deg = cnt_ref[:, 0:1]
    degc = jnp.maximum(deg, 1.0)
    amean = sm / degc
    msq = sq / degc
    astd = jnp.sqrt(jnp.maximum(msq - amean * amean, 0.0) + 1e-5)
    valid = deg > 0.0
    amax = jnp.where(valid, mx_ref[...], 0.0)
    amin = jnp.where(valid, mn_ref[...], 0.0)
    logd = jnp.log(degc + 1.0)
    s_amp = logd / AVG_D_LOG
    s_att = AVG_D_LOG / logd

    wp = wp_ref[...]
    acc = _qdot(h, wp[0:HID, :])
    aggs = (amean, amax, amin, astd)
    for k in range(4):
        a = aggs[k]
        off = HID + k * 3 * HID
        acc += _qdot(a, wp[off:off + HID, :])
        acc += _qdot(a * s_amp, wp[off + HID:off + 2 * HID, :])
        acc += _qdot(a * s_att, wp[off + 2 * HID:off + 3 * HID, :])
    acc = acc + bp_ref[...]
    acc = acc * sn_ref[...]
    rid = b * BN + lax.broadcasted_iota(jnp.int32, (BN, 1), 0)
    accm = jnp.where(rid < N, acc, 0.0)
    ht_ref[...] = acc
    st_ref[0, 0, :] = jnp.sum(accm, axis=0)
    st_ref[0, 1, :] = jnp.sum(accm * accm, axis=0)


def _layer1(h_n, sums, sqs, mxs, mns, cnt_r, wp_i, bp_i, snorm_p):
    f32 = jnp.float32
    full_spec = pl.BlockSpec((BN, HID), lambda b: (b, 0))
    return pl.pallas_call(
        _layer1_body,
        grid=(NB,),
        in_specs=[
            full_spec, full_spec, full_spec, full_spec, full_spec,
            pl.BlockSpec((BN, 16), lambda b: (b, 0)),
            pl.BlockSpec((13 * HID, HID), lambda b: (0, 0)),
            pl.BlockSpec((1, HID), lambda b: (0, 0)),
            pl.BlockSpec((BN, 1), lambda b: (b, 0)),
        ],
        out_specs=[
            pl.BlockSpec((BN, HID), lambda b: (b, 0)),
            pl.BlockSpec((1, 2, HID), lambda b: (b, 0, 0)),
        ],
        out_shape=[jax.ShapeDtypeStruct((NPAD, HID), f32),
                   jax.ShapeDtypeStruct((NB, 2, HID), f32)],
    )(h_n, sums, sqs, mxs, mns, cnt_r, wp_i, bp_i, snorm_p)


def _layer2_body(ht_ref, h_ref, st_ref, g_ref, b_ref, nh_ref, rs_ref):
    b = pl.program_id(0)
    st = st_ref[...]
    mu = jnp.sum(st[:, 0, :], axis=0, keepdims=True) / N
    ex2 = jnp.sum(st[:, 1, :], axis=0, keepdims=True) / N
    var = ex2 - mu * mu
    ht = ht_ref[...]
    htn = g_ref[...] * (ht - mu) / jnp.sqrt(var + 1e-5) + b_ref[...]
    htn = jnp.maximum(htn, 0.0)
    h = h_ref[...] + htn
    nh_ref[...] = h
    rid = b * BN + lax.broadcasted_iota(jnp.int32, (BN, 1), 0)
    hm = jnp.where(rid < N, h, 0.0)
    rs_ref[0, 0, :] = jnp.sum(hm, axis=0)


def _layer2(ht, h_n, stats, gamma_i, beta_i):
    f32 = jnp.float32
    return pl.pallas_call(
        _layer2_body,
        grid=(NB,),
        in_specs=[
            pl.BlockSpec((BN, HID), lambda b: (b, 0)),
            pl.BlockSpec((BN, HID), lambda b: (b, 0)),
            pl.BlockSpec((NB, 2, HID), lambda b: (0, 0, 0)),
            pl.BlockSpec((1, HID), lambda b: (0, 0)),
            pl.BlockSpec((1, HID), lambda b: (0, 0)),
        ],
        out_specs=[
            pl.BlockSpec((BN, HID), lambda b: (b, 0)),
            pl.BlockSpec((1, 1, HID), lambda b: (b, 0, 0)),
        ],
        out_shape=[jax.ShapeDtypeStruct((NPAD, HID), f32),
                   jax.ShapeDtypeStruct((NB, 1, HID), f32)],
    )(ht, h_n, stats, gamma_i, beta_i)


def _readout_body(rs_ref, w0_ref, b0_ref, w1_ref, b1_ref, w2_ref, b2_ref,
                  out_ref):
    hg = jnp.sum(rs_ref[:, 0, :], axis=0, keepdims=True) / N
    x = jnp.maximum(_qdot(hg, w0_ref[...]) + b0_ref[...], 0.0)
    x = jnp.maximum(_qdot(x, w1_ref[...]) + b1_ref[...], 0.0)
    out_ref[...] = _qdot(x, w2_ref[...]) + b2_ref[...]


def _readout(rsum, Wr0, br0, Wr1, br1, Wr2, br2):
    full = lambda shape: pl.BlockSpec(shape, lambda: tuple(0 for _ in shape))
    return pl.pallas_call(
        _readout_body,
        in_specs=[
            full((NB, 1, HID)),
            full((HID, HID // 2)), full((1, HID // 2)),
            full((HID // 2, HID // 4)), full((1, HID // 4)),
            full((HID // 4, 1)), full((1, 1)),
        ],
        out_specs=full((1, 1)),
        out_shape=jax.ShapeDtypeStruct((1, 1), jnp.float32),
    )(rsum, Wr0, br0.reshape(1, -1), Wr1, br1.reshape(1, -1),
      Wr2, br2.reshape(1, -1))


def kernel(h, e, edge_index, snorm_n, snorm_e, emb_h, emb_e, We, be, Wp, bp,
           gamma, beta, Wr0, br0, Wr1, br1, Wr2, br2):
    del snorm_e
    i32 = jnp.int32
    src = edge_index[0].astype(i32)
    dst = edge_index[1].astype(i32)
    e32 = e.astype(i32)

    sdst, spay = lax.sort([dst, src * 4 + e32], num_keys=1)
    ssrc = spay >> 2
    se = spay & 3
    codes = (sdst % NPW) * 4 + se
    pad_i = jnp.zeros((CH,), i32)
    codes_p = jnp.concatenate([codes, pad_i])
    srcs_p = jnp.concatenate([ssrc, pad_i])
    est = jnp.searchsorted(sdst, jnp.arange(NR + 1, dtype=i32) * NPW,
                           side="left").astype(i32)
    est2 = jnp.zeros((NR, 16), i32)
    est2 = est2.at[:, 0].set(est[:NR]).at[:, 1].set(est[1:])
    est2 = est2.reshape(NR * 16)

    h_idx3 = jnp.concatenate([h.astype(i32), jnp.zeros((NPAD - N,), i32)])
    h_idx3 = h_idx3.reshape(NB, 1, BN)
    emb_h_p = jnp.zeros((128, 128), jnp.float32).at[:NUM_ATOM].set(emb_h)
    emb_e_p = jnp.zeros((8, EDIM), jnp.float32).at[:NUM_BOND].set(emb_e)
    snorm_p = jnp.concatenate(
        [snorm_n.astype(jnp.float32),
         jnp.ones((NPAD - N, 1), jnp.float32)])

    h_n = _embed_nodes(h_idx3, emb_h_p)

    tab_all = _edge_tables(emb_e_p, We.reshape(L * EDIM, HID), be)
    tabs = tab_all[:, :NUM_BOND, :].reshape(L, NUM_BOND * HID)

    rsum = None
    for i in range(L):
        sums, sqs, mxs, mns, cnt = _sc_aggregate(
            h_n, codes_p, srcs_p, est2, tabs[i])
        ht, stats = _layer1(h_n,
                            sums.reshape(NPAD, HID), sqs.reshape(NPAD, HID),
                            mxs.reshape(NPAD, HID), mns.reshape(NPAD, HID),
                            cnt.reshape(NPAD, 16),
                            Wp[i], bp[i].reshape(1, HID), snorm_p)
        h_n, rsum = _layer2(ht, h_n, stats,
                            gamma[i].reshape(1, HID),
                            beta[i].reshape(1, HID))

    return _readout(rsum, Wr0, br0, Wr1, br1, Wr2, br2)


# speedup vs baseline: 2.1007x; 1.1555x over previous
import functools

import jax
import jax.numpy as jnp
from jax import lax
from jax.experimental import pallas as pl
from jax.experimental.pallas import tpu as pltpu
from jax.experimental.pallas import tpu_sc as plsc

N = 10000
E = 320000
HID = 128
EDIM = 16
L = 4
NUM_ATOM = 100
NUM_BOND = 4
AVG_D_LOG = 3.4965

NW = 32
NPW = 158
NR = 64
NPAD = NR * NPW
CH = 32
BN = 128
NB = NPAD // BN
DUMMY = NPW * HID


def _qdot(x, w):
    xq = x.astype(jnp.bfloat16).astype(jnp.float32)
    wq = w.astype(jnp.bfloat16).astype(jnp.float32)
    return jnp.dot(xq, wq, preferred_element_type=jnp.float32,
                   precision=lax.Precision.HIGHEST)


def _sc_body(h_hbm, codes_hbm, srcs_hbm, est_hbm, tab_hbm,
             out_sum, out_sq, out_mx, out_mn, out_cnt,
             acc_sum, acc_sq, acc_mx, acc_mn, acc_cnt,
             idx_v, code_v, rows_v, est_v, tab_v, sem):
    cid = lax.axis_index("c")
    sid = lax.axis_index("s")
    wid = sid * 2 + cid

    pltpu.sync_copy(tab_hbm, tab_v)

    zero16 = jnp.zeros((16,), jnp.float32)
    ones16 = jnp.ones((16,), jnp.float32)
    neg16 = jnp.full((16,), -1e30, jnp.float32)
    pos16 = jnp.full((16,), 1e30, jnp.float32)
    iota16 = jnp.arange(16, dtype=jnp.int32)

    for p in range(2):
        r = p * NW + wid
        pltpu.sync_copy(est_hbm.at[pl.ds(r * 16, 16)], est_v)
        ev = est_v[pl.ds(0, 16)]
        e_lo = ev[0]
        e_hi = ev[1]
        a_lo = (e_lo // 8) * 8
        nch = (e_hi - a_lo + CH - 1) // CH

        def init_body(i, _):
            off = i * 16
            acc_sum[pl.ds(off, 16)] = zero16
            acc_sq[pl.ds(off, 16)] = zero16
            acc_mx[pl.ds(off, 16)] = neg16
            acc_mn[pl.ds(off, 16)] = pos16
            return 0
        lax.fori_loop(0, (NPW + 1) * 8, init_body, 0)

        def initc_body(i, _):
            acc_cnt[pl.ds(i * 16, 16)] = zero16
            return 0
        lax.fori_loop(0, NPW + 1, initc_body, 0)

        pltpu.sync_copy(codes_hbm.at[pl.ds(a_lo, CH)],
                        code_v.at[pl.ds(0, CH)])
        pltpu.sync_copy(srcs_hbm.at[pl.ds(a_lo, CH)], idx_v.at[pl.ds(0, CH)])
        pltpu.async_copy(h_hbm.at[idx_v.at[pl.ds(0, CH)]],
                         rows_v.at[pl.ds(0, CH)], sem)

        def chunk_body(k, _):
            po = (k & 1) * CH
            qo = CH - po
            cstart = a_lo + k * CH
            pltpu.make_async_copy(h_hbm.at[idx_v.at[pl.ds(po, CH)]],
                                  rows_v.at[pl.ds(po, CH)], sem).wait()
            cnext = cstart + CH
            pltpu.sync_copy(codes_hbm.at[pl.ds(cnext, CH)],
                            code_v.at[pl.ds(qo, CH)])
            pltpu.sync_copy(srcs_hbm.at[pl.ds(cnext, CH)],
                            idx_v.at[pl.ds(qo, CH)])
            pltpu.async_copy(h_hbm.at[idx_v.at[pl.ds(qo, CH)]],
                             rows_v.at[pl.ds(qo, CH)], sem)
            for g in range(CH // 16):
                code16 = code_v[pl.ds(po + g * 16, 16)]
                jg = cstart + g * 16 + iota16
                vmask = (jg >= e_lo) & (jg < e_hi)
                aoff16 = jnp.where(vmask, (code16 >> 2) << 7, DUMMY)
                eoff16 = (code16 & 3) << 7
                for lane in range(16):
                    ao = aoff16[lane]
                    eo = eoff16[lane]
                    j = po + g * 16 + lane
                    plsc.addupdate(acc_cnt.at[pl.ds((ao >> 3), 16)], ones16)
                    for v in range(8):
                        row = rows_v[j, pl.ds(v * 16, 16)]
                        m = row + tab_v[pl.ds(eo + v * 16, 16)]
                        av = ao + v * 16
                        plsc.addupdate(acc_sum.at[pl.ds(av, 16)], m)
                        plsc.addupdate(acc_sq.at[pl.ds(av, 16)], m * m)
                        acc_mx[pl.ds(av, 16)] = jnp.maximum(
                            acc_mx[pl.ds(av, 16)], m)
                        acc_mn[pl.ds(av, 16)] = jnp.minimum(
                            acc_mn[pl.ds(av, 16)], m)
            return 0
        lax.fori_loop(0, nch, chunk_body, 0)
        peo = (nch & 1) * CH
        pltpu.make_async_copy(h_hbm.at[idx_v.at[pl.ds(peo, CH)]],
                              rows_v.at[pl.ds(peo, CH)], sem).wait()

        npwh = NPW * HID
        base = r * npwh
        pltpu.sync_copy(acc_sum.at[pl.ds(0, npwh)],
                        out_sum.at[pl.ds(base, npwh)])
        pltpu.sync_copy(acc_sq.at[pl.ds(0, npwh)],
                        out_sq.at[pl.ds(base, npwh)])
        pltpu.sync_copy(acc_mx.at[pl.ds(0, npwh)],
                        out_mx.at[pl.ds(base, npwh)])
        pltpu.sync_copy(acc_mn.at[pl.ds(0, npwh)],
                        out_mn.at[pl.ds(base, npwh)])
        pltpu.sync_copy(acc_cnt.at[pl.ds(0, NPW * 16)],
                        out_cnt.at[pl.ds(r * NPW * 16, NPW * 16)])


def _sc_aggregate(h_n, codes_p, srcs_p, est, tab_i):
    mesh = plsc.VectorSubcoreMesh(core_axis_name="c", subcore_axis_name="s")
    f32 = jnp.float32
    agg_t = jax.ShapeDtypeStruct((NPAD * HID,), f32)
    kern = pl.kernel(
        _sc_body,
        mesh=mesh,
        out_type=[agg_t, agg_t, agg_t, agg_t,
                  jax.ShapeDtypeStruct((NPAD * 16,), f32)],
        scratch_types=[
            pltpu.VMEM(((NPW + 1) * HID,), f32),
            pltpu.VMEM(((NPW + 1) * HID,), f32),
            pltpu.VMEM(((NPW + 1) * HID,), f32),
            pltpu.VMEM(((NPW + 1) * HID,), f32),
            pltpu.VMEM(((NPW + 1) * 16,), f32),
            pltpu.VMEM((2 * CH,), jnp.int32),
            pltpu.VMEM((2 * CH,), jnp.int32),
            pltpu.VMEM((2 * CH, HID), f32),
            pltpu.VMEM((16,), jnp.int32),
            pltpu.VMEM((NUM_BOND * HID,), f32),
            pltpu.SemaphoreType.DMA,
        ],
    )
    return kern(h_n, codes_p, srcs_p, est, tab_i)


def _embed_body(hid_ref, emb_ref, h_ref):
    ids = hid_ref[0, 0, :]
    col = lax.broadcasted_iota(jnp.int32, (BN, 128), 1)
    oh = jnp.where(ids[:, None] == col, 1.0, 0.0).astype(jnp.float32)
    h_ref[...] = jnp.dot(oh, emb_ref[...], preferred_element_type=jnp.float32,
                     precision=lax.Precision.HIGHEST)


def _embed_nodes(h_idx3, emb_h_p):
    return pl.pallas_call(
        _embed_body,
        grid=(NB,),
        in_specs=[
            pl.BlockSpec((1, 1, BN), lambda b: (b, 0, 0)),
            pl.BlockSpec((128, 128), lambda b: (0, 0)),
        ],
        out_specs=pl.BlockSpec((BN, HID), lambda b: (b, 0)),
        out_shape=jax.ShapeDtypeStruct((NPAD, HID), jnp.float32),
    )(h_idx3, emb_h_p)


def _tab_body(embe_ref, We_ref, be_ref, tab_ref):
    embe = embe_ref[...]
    for i in range(L):
        w = We_ref[pl.ds(i * EDIM, EDIM), :]
        t = _qdot(embe, w)
        tab_ref[i, :, :] = t + be_ref[i, :][None, :]


def _edge_tables(emb_e_p, We_r, be):
    return pl.pallas_call(
        _tab_body,
        grid=(1,),
        in_specs=[
            pl.BlockSpec((8, EDIM), lambda b: (0, 0)),
            pl.BlockSpec((L * EDIM, HID), lambda b: (0, 0)),
            pl.BlockSpec((L, HID), lambda b: (0, 0)),
        ],
        out_specs=pl.BlockSpec((L, 8, HID), lambda b: (0, 0, 0)),
        out_shape=jax.ShapeDtypeStruct((L, 8, HID), jnp.float32),
    )(emb_e_p, We_r, be)


def _layer1_body(h_ref, sm_ref, sq_ref, mx_ref, mn_ref,
                 cnt_ref, wp_ref, bp_ref, sn_ref, ht_ref, st_ref):
    b = pl.program_id(0)
    h = h_ref[...]
    sm = sm_ref[...]
    sq = sq_ref[...]
    deg = cnt_ref[:, 0:1]
    degc = jnp.maximum(deg, 1.0)
    amean = sm / degc
    msq = sq / degc
    astd = jnp.sqrt(jnp.maximum(msq - amean * amean, 0.0) + 1e-5)
    valid = deg > 0.0
    amax = jnp.where(valid, mx_ref[...], 0.0)
    amin = jnp.where(valid, mn_ref[...], 0.0)
    logd = jnp.log(degc + 1.0)
    s_amp = logd / AVG_D_LOG
    s_att = AVG_D_LOG / logd

    wp = wp_ref[...]
    acc = _qdot(h, wp[0:HID, :])
    aggs = (amean, amax, amin, astd)
    for k in range(4):
        a = aggs[k]
        off = HID + k * 3 * HID
        acc += _qdot(a, wp[off:off + HID, :])
        acc += _qdot(a * s_amp, wp[off + HID:off + 2 * HID, :])
        acc += _qdot(a * s_att, wp[off + 2 * HID:off + 3 * HID, :])
    acc = acc + bp_ref[...]
    acc = acc * sn_ref[...]
    rid = b * BN + lax.broadcasted_iota(jnp.int32, (BN, 1), 0)
    accm = jnp.where(rid < N, acc, 0.0)
    ht_ref[...] = acc
    st_ref[0, 0, :] = jnp.sum(accm, axis=0)
    st_ref[0, 1, :] = jnp.sum(accm * accm, axis=0)


def _layer1(h_n, sums, sqs, mxs, mns, cnt_r, wp_i, bp_i, snorm_p):
    f32 = jnp.float32
    full_spec = pl.BlockSpec((BN, HID), lambda b: (b, 0))
    return pl.pallas_call(
        _layer1_body,
        grid=(NB,),
        in_specs=[
            full_spec, full_spec, full_spec, full_spec, full_spec,
            pl.BlockSpec((BN, 16), lambda b: (b, 0)),
            pl.BlockSpec((13 * HID, HID), lambda b: (0, 0)),
            pl.BlockSpec((1, HID), lambda b: (0, 0)),
            pl.BlockSpec((BN, 1), lambda b: (b, 0)),
        ],
        out_specs=[
            pl.BlockSpec((BN, HID), lambda b: (b, 0)),
            pl.BlockSpec((1, 2, HID), lambda b: (b, 0, 0)),
        ],
        out_shape=[jax.ShapeDtypeStruct((NPAD, HID), f32),
                   jax.ShapeDtypeStruct((NB, 2, HID), f32)],
    )(h_n, sums, sqs, mxs, mns, cnt_r, wp_i, bp_i, snorm_p)


def _layer2_body(ht_ref, h_ref, st_ref, g_ref, b_ref, nh_ref, rs_ref):
    b = pl.program_id(0)
    st = st_ref[...]
    mu = jnp.sum(st[:, 0, :], axis=0, keepdims=True) / N
    ex2 = jnp.sum(st[:, 1, :], axis=0, keepdims=True) / N
    var = ex2 - mu * mu
    ht = ht_ref[...]
    htn = g_ref[...] * (ht - mu) / jnp.sqrt(var + 1e-5) + b_ref[...]
    htn = jnp.maximum(htn, 0.0)
    h = h_ref[...] + htn
    nh_ref[...] = h
    rid = b * BN + lax.broadcasted_iota(jnp.int32, (BN, 1), 0)
    hm = jnp.where(rid < N, h, 0.0)
    rs_ref[0, 0, :] = jnp.sum(hm, axis=0)


def _layer2(ht, h_n, stats, gamma_i, beta_i):
    f32 = jnp.float32
    return pl.pallas_call(
        _layer2_body,
        grid=(NB,),
        in_specs=[
            pl.BlockSpec((BN, HID), lambda b: (b, 0)),
            pl.BlockSpec((BN, HID), lambda b: (b, 0)),
            pl.BlockSpec((NB, 2, HID), lambda b: (0, 0, 0)),
            pl.BlockSpec((1, HID), lambda b: (0, 0)),
            pl.BlockSpec((1, HID), lambda b: (0, 0)),
        ],
        out_specs=[
            pl.BlockSpec((BN, HID), lambda b: (b, 0)),
            pl.BlockSpec((1, 1, HID), lambda b: (b, 0, 0)),
        ],
        out_shape=[jax.ShapeDtypeStruct((NPAD, HID), f32),
                   jax.ShapeDtypeStruct((NB, 1, HID), f32)],
    )(ht, h_n, stats, gamma_i, beta_i)


def _readout_body(rs_ref, w0_ref, b0_ref, w1_ref, b1_ref, w2_ref, b2_ref,
                  out_ref):
    hg = jnp.sum(rs_ref[:, 0, :], axis=0, keepdims=True) / N
    x = jnp.maximum(_qdot(hg, w0_ref[...]) + b0_ref[...], 0.0)
    x = jnp.maximum(_qdot(x, w1_ref[...]) + b1_ref[...], 0.0)
    out_ref[...] = _qdot(x, w2_ref[...]) + b2_ref[...]


def _readout(rsum, Wr0, br0, Wr1, br1, Wr2, br2):
    full = lambda shape: pl.BlockSpec(shape, lambda: tuple(0 for _ in shape))
    return pl.pallas_call(
        _readout_body,
        in_specs=[
            full((NB, 1, HID)),
            full((HID, HID // 2)), full((1, HID // 2)),
            full((HID // 2, HID // 4)), full((1, HID // 4)),
            full((HID // 4, 1)), full((1, 1)),
        ],
        out_specs=full((1, 1)),
        out_shape=jax.ShapeDtypeStruct((1, 1), jnp.float32),
    )(rsum, Wr0, br0.reshape(1, -1), Wr1, br1.reshape(1, -1),
      Wr2, br2.reshape(1, -1))


def kernel(h, e, edge_index, snorm_n, snorm_e, emb_h, emb_e, We, be, Wp, bp,
           gamma, beta, Wr0, br0, Wr1, br1, Wr2, br2):
    del snorm_e
    i32 = jnp.int32
    src = edge_index[0].astype(i32)
    dst = edge_index[1].astype(i32)
    e32 = e.astype(i32)

    sdst, spay = lax.sort([dst, src * 4 + e32], num_keys=1)
    ssrc = spay >> 2
    se = spay & 3
    codes = (sdst % NPW) * 4 + se
    pad_i = jnp.zeros((4 * CH,), i32)
    codes_p = jnp.concatenate([codes, pad_i])
    srcs_p = jnp.concatenate([ssrc, pad_i])
    est = jnp.searchsorted(sdst, jnp.arange(NR + 1, dtype=i32) * NPW,
                           side="left").astype(i32)
    est2 = jnp.zeros((NR, 16), i32)
    est2 = est2.at[:, 0].set(est[:NR]).at[:, 1].set(est[1:])
    est2 = est2.reshape(NR * 16)

    h_idx3 = jnp.concatenate([h.astype(i32), jnp.zeros((NPAD - N,), i32)])
    h_idx3 = h_idx3.reshape(NB, 1, BN)
    emb_h_p = jnp.zeros((128, 128), jnp.float32).at[:NUM_ATOM].set(emb_h)
    emb_e_p = jnp.zeros((8, EDIM), jnp.float32).at[:NUM_BOND].set(emb_e)
    snorm_p = jnp.concatenate(
        [snorm_n.astype(jnp.float32),
         jnp.ones((NPAD - N, 1), jnp.float32)])

    h_n = _embed_nodes(h_idx3, emb_h_p)

    tab_all = _edge_tables(emb_e_p, We.reshape(L * EDIM, HID), be)
    tabs = tab_all[:, :NUM_BOND, :].reshape(L, NUM_BOND * HID)

    rsum = None
    for i in range(L):
        sums, sqs, mxs, mns, cnt = _sc_aggregate(
            h_n, codes_p, srcs_p, est2, tabs[i])
        ht, stats = _layer1(h_n,
                            sums.reshape(NPAD, HID), sqs.reshape(NPAD, HID),
                            mxs.reshape(NPAD, HID), mns.reshape(NPAD, HID),
                            cnt.reshape(NPAD, 16),
                            Wp[i], bp[i].reshape(1, HID), snorm_p)
        h_n, rsum = _layer2(ht, h_n, stats,
                            gamma[i].reshape(1, HID),
                            beta[i].reshape(1, HID))

    return _readout(rsum, Wr0, br0, Wr1, br1, Wr2, br2)
